# baseline (speedup 1.0000x reference)
import functools

import jax
import jax.numpy as jnp
from jax import lax
from jax.experimental import pallas as pl
from jax.experimental.pallas import tpu as pltpu
from jax.experimental.pallas import tpu_sc as plsc

NC = 2
NS = 16
NW = NC * NS


def _sc_gather_sum(u_pk, nbr_flat, entity, emb):
    V, DW = u_pk.shape
    N = entity.shape[0]
    K = nbr_flat.shape[0] // N
    E = emb.shape[1]
    TPW = N // NW
    BCH = 128
    NT = DW // 16

    mesh = plsc.VectorSubcoreMesh(
        core_axis_name="c", subcore_axis_name="s",
        num_cores=NC, num_subcores=NS)

    @functools.partial(
        pl.kernel,
        out_type=(jax.ShapeDtypeStruct((N, 2 * DW), jnp.float32),
                  jax.ShapeDtypeStruct((N, E), jnp.float32)),
        mesh=mesh,
        scratch_types=[
            pltpu.VMEM((TPW * K,), jnp.int32),
            pltpu.VMEM((TPW,), jnp.int32),
            pltpu.VMEM((2, K, DW), jnp.int32),
            pltpu.VMEM((2, 2, 2 * DW), jnp.float32),
            pltpu.VMEM((BCH, E), jnp.float32),
            pltpu.SemaphoreType.DMA,
            pltpu.SemaphoreType.DMA,
            pltpu.SemaphoreType.DMA,
            pltpu.SemaphoreType.DMA,
            pltpu.SemaphoreType.DMA,
        ],
    )
    def k(u_hbm, nbr_hbm, ent_hbm, emb_hbm, ur_hbm, base_hbm,
          idx_v, ent_v, rows_v, out_v, base_v, g0, g1, so0, so1, sb):
        wid = lax.axis_index("s") * NC + lax.axis_index("c")
        t0 = wid * TPW
        gsem = (g0, g1)
        osem = (so0, so1)

        pltpu.sync_copy(nbr_hbm.at[pl.ds(t0 * K, TPW * K)], idx_v)
        pltpu.sync_copy(ent_hbm.at[pl.ds(t0, TPW)], ent_v)

        def gather(lt, b):
            return pltpu.make_async_copy(
                u_hbm.at[idx_v.at[pl.ds(lt * K, K)]], rows_v.at[b], gsem[b])

        def out_dma(ob, row):
            return pltpu.make_async_copy(
                out_v.at[ob], ur_hbm.at[pl.ds(row, 2)], osem[ob])

        def load_pair(b, kk, j):
            word = rows_v[b, kk, pl.ds(j * 16, 16)]
            lo = lax.bitcast_convert_type(word << 16, jnp.float32)
            hi = lax.bitcast_convert_type(word, jnp.float32)
            return lo, hi

        def reduce_rows(b, ob):
            def body(kk, acc):
                new = []
                for j in range(NT):
                    lo, hi = load_pair(b, kk, j)
                    new.append(acc[2 * j] + lo)
                    new.append(acc[2 * j + 1] + hi)
                return tuple(new)
            acc = []
            for j in range(NT):
                lo, hi = load_pair(b, 0, j)
                acc.extend((lo, hi))
            acc = lax.fori_loop(1, K, body, tuple(acc))
            for j in range(NT):
                out_v[ob, b, pl.ds(j * 32, 16)] = acc[2 * j]
                out_v[ob, b, pl.ds(j * 32 + 16, 16)] = acc[2 * j + 1]

        gather(0, 0).start()

        def outer(cc, carry):
            for ob in range(2):
                @pl.when(cc > 0)
                def _wait_prev_out():
                    out_dma(ob, 0).wait()
                for b in range(2):
                    lt = cc * 4 + ob * 2 + b
                    nxt = jnp.minimum(lt + 1, TPW - 1)
                    gather(nxt, 1 - b).start()
                    gather(lt, b).wait()
                    reduce_rows(b, ob)
                out_dma(ob, t0 + cc * 4 + ob * 2).start()
            return carry

        lax.fori_loop(0, TPW // 4, outer, 0)

        gather(0, 0).wait()
        out_dma(0, 0).wait()
        out_dma(1, 0).wait()

        for c in range(TPW // BCH):
            pltpu.async_copy(
                emb_hbm.at[ent_v.at[pl.ds(c * BCH, BCH)]], base_v, sb).wait()
            pltpu.sync_copy(base_v, base_hbm.at[pl.ds(t0 + c * BCH, BCH)])

    return k(u_pk, nbr_flat, entity, emb)


def _tc_dense(ur_sum, ea, base, Wbig, wT, M2, Sel, Sel2, Rep, Til,
              K, R1, U, A):
    N, D = ur_sum.shape
    E = base.shape[1]
    BN = 256
    inv_k = 1.0 / K

    def body(ur_ref, ea_ref, base_ref, Wbig_ref, wT_ref, M2_ref,
             Sel_ref, Sel2_ref, Rep_ref, Til_ref, out_ref):
        urb = ur_ref[...]
        eab = ea_ref[...]
        Wb = Wbig_ref[...]
        Selm = Sel_ref[...]
        earep = jnp.dot(eab, Rep_ref[...],
                        preferred_element_type=jnp.float32)
        wrrep = jnp.dot(eab, wT_ref[...],
                        preferred_element_type=jnp.float32)

        qs = []
        for r in range(R1):
            ur_r = urb[:, r * U:(r + 1) * U]
            Pr = jnp.dot(ur_r, Wb,
                         preferred_element_type=jnp.float32)
            Qr = jnp.dot(Pr * earep, Selm,
                         preferred_element_type=jnp.float32)
            qs.append(jnp.tanh(Qr))
        tq = jnp.concatenate(qs, axis=1)

        scores = jnp.dot(tq * wrrep, Sel2_ref[...],
                         preferred_element_type=jnp.float32)
        m = jnp.max(scores, axis=1, keepdims=True)
        ex = jnp.exp(scores - m)
        att = ex / jnp.sum(ex, axis=1, keepdims=True)
        attk = att * inv_k

        attrep = jnp.dot(attk, Rep_ref[...],
                         preferred_element_type=jnp.float32)
        vv = jnp.dot(attrep * urb, Selm,
                     preferred_element_type=jnp.float32)
        T = jnp.dot(vv, Til_ref[...],
                    preferred_element_type=jnp.float32) * earep
        agg = jnp.dot(T, M2_ref[...],
                      preferred_element_type=jnp.float32)
        out = base_ref[...] + agg
        nrm = jnp.sqrt(jnp.sum(out * out, axis=1, keepdims=True))
        out_ref[...] = out / jnp.maximum(nrm, 1e-12)

    grid = (N // BN,)
    fixed = lambda shape: pl.BlockSpec(shape, lambda i: (0, 0))
    return pl.pallas_call(
        body,
        grid=grid,
        in_specs=[
            pl.BlockSpec((BN, D), lambda i: (i, 0)),
            pl.BlockSpec((BN, R1), lambda i: (i, 0)),
            pl.BlockSpec((BN, E), lambda i: (i, 0)),
            fixed(Wbig.shape),
            fixed(wT.shape),
            fixed(M2.shape),
            fixed(Sel.shape),
            fixed(Sel2.shape),
            fixed(Rep.shape),
            fixed(Til.shape),
        ],
        out_specs=pl.BlockSpec((BN, E), lambda i: (i, 0)),
        out_shape=jax.ShapeDtypeStruct((N, E), jnp.float32),
    )(ur_sum, ea, base, Wbig, wT, M2, Sel, Sel2, Rep, Til)


def kernel(entity, edge_attr, entity_neighbours, entity_embeddings, u, W, w, M):
    N, K = entity_neighbours.shape
    V, R1, U = u.shape
    A = W.shape[2]
    E = entity_embeddings.shape[1]
    D = R1 * U

    u_pk = lax.bitcast_convert_type(
        u.astype(jnp.bfloat16).reshape(V, D // 2, 2), jnp.int32)
    nbr_flat = entity_neighbours.reshape(N * K)

    ur_sum, base = _sc_gather_sum(u_pk, nbr_flat, entity, entity_embeddings)

    u_perm = jnp.arange(U).reshape(U // 2, 2).T.reshape(U)
    Wbig = (W.transpose(1, 0, 2).reshape(U, R1 * A) * (1.0 / K))[u_perm]
    M2 = M.reshape(R1, U, E)[:, u_perm, :].reshape(R1 * U, E)
    eyeR = jnp.eye(R1, dtype=jnp.float32)
    eyeU = jnp.eye(U, dtype=jnp.float32)
    Sel = jnp.tile(eyeU, (R1, 1))
    Sel2 = jnp.repeat(eyeR, A, axis=0)
    Rep = Sel2.T
    Til = jnp.tile(eyeU, (1, R1))
    wT = w @ Til

    CH = 4
    NCk = N // CH
    outs = []
    for c in range(CH):
        sl = slice(c * NCk, (c + 1) * NCk)
        ur_c, base_c = _sc_gather_sum(
            u_pk, nbr_flat[c * NCk * K:(c + 1) * NCk * K],
            entity[sl], entity_embeddings)
        outs.append(_tc_dense(ur_c, edge_attr[sl], base_c, Wbig, wT, M2,
                              Sel, Sel2, Rep, Til, K, R1, U, A))
    return jnp.concatenate(outs, axis=0)


# speedup vs baseline: 1.8737x; 1.8737x over previous
import functools

import jax
import jax.numpy as jnp
from jax import lax
from jax.experimental import pallas as pl
from jax.experimental.pallas import tpu as pltpu
from jax.experimental.pallas import tpu_sc as plsc

NC = 2
NS = 16
NW = NC * NS


def _sc_gather_sum(u_pk, nbr_flat, entity, emb):
    V, DW = u_pk.shape
    N = entity.shape[0]
    K = nbr_flat.shape[0] // N
    E = emb.shape[1]
    TPW = N // NW
    BCH = 128
    NT = DW // 16

    mesh = plsc.VectorSubcoreMesh(
        core_axis_name="c", subcore_axis_name="s",
        num_cores=NC, num_subcores=NS)

    @functools.partial(
        pl.kernel,
        out_type=(jax.ShapeDtypeStruct((N, 2 * DW), jnp.float32),
                  jax.ShapeDtypeStruct((N, E), jnp.float32)),
        mesh=mesh,
        scratch_types=[
            pltpu.VMEM((TPW * K,), jnp.int32),
            pltpu.VMEM((TPW,), jnp.int32),
            pltpu.VMEM((2, K, DW), jnp.int32),
            pltpu.VMEM((2, 2, 2 * DW), jnp.float32),
            pltpu.VMEM((BCH, E), jnp.float32),
            pltpu.SemaphoreType.DMA,
            pltpu.SemaphoreType.DMA,
            pltpu.SemaphoreType.DMA,
            pltpu.SemaphoreType.DMA,
            pltpu.SemaphoreType.DMA,
        ],
    )
    def k(u_hbm, nbr_hbm, ent_hbm, emb_hbm, ur_hbm, base_hbm,
          idx_v, ent_v, rows_v, out_v, base_v, g0, g1, so0, so1, sb):
        wid = lax.axis_index("s") * NC + lax.axis_index("c")
        t0 = wid * TPW
        gsem = (g0, g1)
        osem = (so0, so1)

        pltpu.sync_copy(nbr_hbm.at[pl.ds(t0 * K, TPW * K)], idx_v)
        pltpu.sync_copy(ent_hbm.at[pl.ds(t0, TPW)], ent_v)

        def gather(lt, b):
            return pltpu.make_async_copy(
                u_hbm.at[idx_v.at[pl.ds(lt * K, K)]], rows_v.at[b], gsem[b])

        def out_dma(ob, row):
            return pltpu.make_async_copy(
                out_v.at[ob], ur_hbm.at[pl.ds(row, 2)], osem[ob])

        def load_pair(b, kk, j):
            word = rows_v[b, kk, pl.ds(j * 16, 16)]
            lo = lax.bitcast_convert_type(word << 16, jnp.float32)
            hi = lax.bitcast_convert_type(word, jnp.float32)
            return lo, hi

        def reduce_rows(b, ob):
            def body(kk, acc):
                new = []
                for j in range(NT):
                    lo, hi = load_pair(b, kk, j)
                    new.append(acc[2 * j] + lo)
                    new.append(acc[2 * j + 1] + hi)
                return tuple(new)
            acc = []
            for j in range(NT):
                lo, hi = load_pair(b, 0, j)
                acc.extend((lo, hi))
            acc = lax.fori_loop(1, K, body, tuple(acc))
            for j in range(NT):
                out_v[ob, b, pl.ds(j * 16, 16)] = acc[2 * j]
                out_v[ob, b, pl.ds(DW + j * 16, 16)] = acc[2 * j + 1]

        gather(0, 0).start()

        def outer(cc, carry):
            for ob in range(2):
                @pl.when(cc > 0)
                def _wait_prev_out():
                    out_dma(ob, 0).wait()
                for b in range(2):
                    lt = cc * 4 + ob * 2 + b
                    nxt = jnp.minimum(lt + 1, TPW - 1)
                    gather(nxt, 1 - b).start()
                    gather(lt, b).wait()
                    reduce_rows(b, ob)
                out_dma(ob, t0 + cc * 4 + ob * 2).start()
            return carry

        lax.fori_loop(0, TPW // 4, outer, 0)

        gather(0, 0).wait()
        out_dma(0, 0).wait()
        out_dma(1, 0).wait()

        for c in range(TPW // BCH):
            pltpu.async_copy(
                emb_hbm.at[ent_v.at[pl.ds(c * BCH, BCH)]], base_v, sb).wait()
            pltpu.sync_copy(base_v, base_hbm.at[pl.ds(t0 + c * BCH, BCH)])

    return k(u_pk, nbr_flat, entity, emb)


def _tc_dense(ur_sum, ea, base, Wbig, wT, M2, Sel, Sel2, Rep, Til,
              K, R1, U, A):
    N, D = ur_sum.shape
    E = base.shape[1]
    BN = 512
    inv_k = 1.0 / K

    def body(ur_ref, ea_ref, base_ref, Wbig_ref, wT_ref, M2_ref,
             Sel_ref, Sel2_ref, Rep_ref, Til_ref, out_ref):
        urb = ur_ref[...]
        eab = ea_ref[...]
        Wb = Wbig_ref[...]
        Selm = Sel_ref[...]
        earep = jnp.dot(eab, Rep_ref[...],
                        preferred_element_type=jnp.float32)
        wrrep = jnp.dot(eab, wT_ref[...],
                        preferred_element_type=jnp.float32)

        qs = []
        for r in range(R1):
            ur_r = urb[:, r * U:(r + 1) * U]
            Pr = jnp.dot(ur_r, Wb,
                         preferred_element_type=jnp.float32)
            Qr = jnp.dot(Pr * earep, Selm,
                         preferred_element_type=jnp.float32)
            qs.append(Qr)
        tq = jnp.tanh(jnp.concatenate(qs, axis=1))

        scores = jnp.dot(tq * wrrep, Sel2_ref[...],
                         preferred_element_type=jnp.float32)
        m = jnp.max(scores, axis=1, keepdims=True)
        ex = jnp.exp(scores - m)
        att = ex / jnp.sum(ex, axis=1, keepdims=True)
        attk = att * inv_k

        attrep = jnp.dot(attk, Rep_ref[...],
                         preferred_element_type=jnp.float32)
        vv = jnp.dot(attrep * urb, Selm,
                     preferred_element_type=jnp.float32)
        T = jnp.dot(vv, Til_ref[...],
                    preferred_element_type=jnp.float32) * earep
        agg = jnp.dot(T, M2_ref[...],
                      preferred_element_type=jnp.float32)
        out = base_ref[...] + agg
        nrm = jnp.sqrt(jnp.sum(out * out, axis=1, keepdims=True))
        out_ref[...] = out / jnp.maximum(nrm, 1e-12)

    grid = (N // BN,)
    fixed = lambda shape: pl.BlockSpec(shape, lambda i: (0, 0))
    return pl.pallas_call(
        body,
        grid=grid,
        in_specs=[
            pl.BlockSpec((BN, D), lambda i: (i, 0)),
            pl.BlockSpec((BN, R1), lambda i: (i, 0)),
            pl.BlockSpec((BN, E), lambda i: (i, 0)),
            fixed(Wbig.shape),
            fixed(wT.shape),
            fixed(M2.shape),
            fixed(Sel.shape),
            fixed(Sel2.shape),
            fixed(Rep.shape),
            fixed(Til.shape),
        ],
        out_specs=pl.BlockSpec((BN, E), lambda i: (i, 0)),
        out_shape=jax.ShapeDtypeStruct((N, E), jnp.float32),
    )(ur_sum, ea, base, Wbig, wT, M2, Sel, Sel2, Rep, Til)


def kernel(entity, edge_attr, entity_neighbours, entity_embeddings, u, W, w, M):
    N, K = entity_neighbours.shape
    V, R1, U = u.shape
    A = W.shape[2]
    E = entity_embeddings.shape[1]
    D = R1 * U

    HR = R1 // 2
    lo = lax.bitcast_convert_type(
        u[:, :HR, :].astype(jnp.bfloat16), jnp.uint16).astype(jnp.uint32)
    hi = lax.bitcast_convert_type(
        u[:, HR:, :].astype(jnp.bfloat16), jnp.uint16).astype(jnp.uint32)
    u_pk = lax.bitcast_convert_type(lo | (hi << 16),
                                    jnp.int32).reshape(V, D // 2)
    nbr_flat = entity_neighbours.reshape(N * K)

    ur_sum, base = _sc_gather_sum(u_pk, nbr_flat, entity, entity_embeddings)

    Wbig = W.transpose(1, 0, 2).reshape(U, R1 * A) * (1.0 / K)
    M2 = M.reshape(R1 * U, E)
    eyeR = jnp.eye(R1, dtype=jnp.float32)
    eyeU = jnp.eye(U, dtype=jnp.float32)
    Sel = jnp.tile(eyeU, (R1, 1))
    Sel2 = jnp.repeat(eyeR, A, axis=0)
    Rep = Sel2.T
    Til = jnp.tile(eyeU, (1, R1))
    wT = w @ Til

    CH = 4
    NCk = N // CH
    outs = []
    for c in range(CH):
        sl = slice(c * NCk, (c + 1) * NCk)
        ur_c, base_c = _sc_gather_sum(
            u_pk, nbr_flat[c * NCk * K:(c + 1) * NCk * K],
            entity[sl], entity_embeddings)
        outs.append(_tc_dense(ur_c, edge_attr[sl], base_c, Wbig, wT, M2,
                              Sel, Sel2, Rep, Til, K, R1, U, A))
    return jnp.concatenate(outs, axis=0)


# speedup vs baseline: 2.0809x; 1.1106x over previous
import functools

import jax
import jax.numpy as jnp
from jax import lax
from jax.experimental import pallas as pl
from jax.experimental.pallas import tpu as pltpu
from jax.experimental.pallas import tpu_sc as plsc

NC = 2
NS = 16
NW = NC * NS


def _sc_gather_sum(u_pk, nbr_flat, entity, emb):
    V, DW = u_pk.shape
    N = entity.shape[0]
    K = nbr_flat.shape[0] // N
    E = emb.shape[1]
    TPW = N // NW
    BCH = 128
    NT = DW // 16

    mesh = plsc.VectorSubcoreMesh(
        core_axis_name="c", subcore_axis_name="s",
        num_cores=NC, num_subcores=NS)

    @functools.partial(
        pl.kernel,
        out_type=(jax.ShapeDtypeStruct((N, 2 * DW), jnp.float32),
                  jax.ShapeDtypeStruct((N, E), jnp.float32)),
        mesh=mesh,
        scratch_types=[
            pltpu.VMEM((TPW * K,), jnp.int32),
            pltpu.VMEM((TPW,), jnp.int32),
            pltpu.VMEM((4, K, DW), jnp.int32),
            pltpu.VMEM((2, 2, 2 * DW), jnp.float32),
            pltpu.VMEM((BCH, E), jnp.float32),
            pltpu.SemaphoreType.DMA,
            pltpu.SemaphoreType.DMA,
            pltpu.SemaphoreType.DMA,
            pltpu.SemaphoreType.DMA,
            pltpu.SemaphoreType.DMA,
            pltpu.SemaphoreType.DMA,
            pltpu.SemaphoreType.DMA,
        ],
    )
    def k(u_hbm, nbr_hbm, ent_hbm, emb_hbm, ur_hbm, base_hbm,
          idx_v, ent_v, rows_v, out_v, base_v, g0, g1, g2, g3,
          so0, so1, sb):
        wid = lax.axis_index("s") * NC + lax.axis_index("c")
        t0 = wid * TPW
        gsem = (g0, g1, g2, g3)
        osem = (so0, so1)

        pltpu.sync_copy(nbr_hbm.at[pl.ds(t0 * K, TPW * K)], idx_v)
        pltpu.sync_copy(ent_hbm.at[pl.ds(t0, TPW)], ent_v)

        def gather(lt, b):
            return pltpu.make_async_copy(
                u_hbm.at[idx_v.at[pl.ds(lt * K, K)]], rows_v.at[b], gsem[b])

        def out_dma(ob, row):
            return pltpu.make_async_copy(
                out_v.at[ob], ur_hbm.at[pl.ds(row, 2)], osem[ob])

        def load_pair(gb, kk, j):
            word = rows_v[gb, kk, pl.ds(j * 16, 16)]
            lo = lax.bitcast_convert_type(word << 16, jnp.float32)
            hi = lax.bitcast_convert_type(word, jnp.float32)
            return lo, hi

        def reduce_rows(gb, b, ob):
            def body(kk, acc):
                new = []
                for j in range(NT):
                    lo, hi = load_pair(gb, kk, j)
                    new.append(acc[2 * j] + lo)
                    new.append(acc[2 * j + 1] + hi)
                return tuple(new)
            acc = []
            for j in range(NT):
                lo, hi = load_pair(gb, 0, j)
                acc.extend((lo, hi))
            acc = lax.fori_loop(1, K, body, tuple(acc))
            for j in range(NT):
                out_v[ob, b, pl.ds(j * 16, 16)] = acc[2 * j]
                out_v[ob, b, pl.ds(DW + j * 16, 16)] = acc[2 * j + 1]

        gather(0, 0).start()
        gather(1, 1).start()
        gather(2, 2).start()

        def outer(cc, carry):
            for ob in range(2):
                @pl.when(cc > 0)
                def _wait_prev_out():
                    out_dma(ob, 0).wait()
                for b in range(2):
                    gb = ob * 2 + b
                    lt = cc * 4 + gb
                    nxt = jnp.minimum(lt + 3, TPW - 1)
                    gather(nxt, (gb + 3) % 4).start()
                    gather(lt, gb).wait()
                    reduce_rows(gb, b, ob)
                out_dma(ob, t0 + cc * 4 + ob * 2).start()
            return carry

        lax.fori_loop(0, TPW // 4, outer, 0)

        gather(0, 0).wait()
        gather(0, 1).wait()
        gather(0, 2).wait()
        out_dma(0, 0).wait()
        out_dma(1, 0).wait()

        for c in range(TPW // BCH):
            pltpu.async_copy(
                emb_hbm.at[ent_v.at[pl.ds(c * BCH, BCH)]], base_v, sb).wait()
            pltpu.sync_copy(base_v, base_hbm.at[pl.ds(t0 + c * BCH, BCH)])

    return k(u_pk, nbr_flat, entity, emb)


def _tc_dense(ur_sum, ea, base, Wbig, wT, M2, Sel, Sel2, Rep, Til,
              K, R1, U, A):
    N, D = ur_sum.shape
    E = base.shape[1]
    BN = 512
    inv_k = 1.0 / K

    def body(ur_ref, ea_ref, base_ref, Wbig_ref, wT_ref, M2_ref,
             Sel_ref, Sel2_ref, Rep_ref, Til_ref, out_ref):
        urb = ur_ref[...]
        eab = ea_ref[...]
        Wb = Wbig_ref[...]
        Selm = Sel_ref[...]
        earep = jnp.dot(eab, Rep_ref[...],
                        preferred_element_type=jnp.float32)
        wrrep = jnp.dot(eab, wT_ref[...],
                        preferred_element_type=jnp.float32)

        qs = []
        for r in range(R1):
            ur_r = urb[:, r * U:(r + 1) * U]
            Pr = jnp.dot(ur_r, Wb,
                         preferred_element_type=jnp.float32)
            Qr = jnp.dot(Pr * earep, Selm,
                         preferred_element_type=jnp.float32)
            qs.append(Qr)
        tq = jnp.tanh(jnp.concatenate(qs, axis=1))

        scores = jnp.dot(tq * wrrep, Sel2_ref[...],
                         preferred_element_type=jnp.float32)
        m = jnp.max(scores, axis=1, keepdims=True)
        ex = jnp.exp(scores - m)
        att = ex / jnp.sum(ex, axis=1, keepdims=True)
        attk = att * inv_k

        attrep = jnp.dot(attk, Rep_ref[...],
                         preferred_element_type=jnp.float32)
        vv = jnp.dot(attrep * urb, Selm,
                     preferred_element_type=jnp.float32)
        T = jnp.dot(vv, Til_ref[...],
                    preferred_element_type=jnp.float32) * earep
        agg = jnp.dot(T, M2_ref[...],
                      preferred_element_type=jnp.float32)
        out = base_ref[...] + agg
        nrm = jnp.sqrt(jnp.sum(out * out, axis=1, keepdims=True))
        out_ref[...] = out / jnp.maximum(nrm, 1e-12)

    grid = (N // BN,)
    fixed = lambda shape: pl.BlockSpec(shape, lambda i: (0, 0))
    return pl.pallas_call(
        body,
        grid=grid,
        in_specs=[
            pl.BlockSpec((BN, D), lambda i: (i, 0)),
            pl.BlockSpec((BN, R1), lambda i: (i, 0)),
            pl.BlockSpec((BN, E), lambda i: (i, 0)),
            fixed(Wbig.shape),
            fixed(wT.shape),
            fixed(M2.shape),
            fixed(Sel.shape),
            fixed(Sel2.shape),
            fixed(Rep.shape),
            fixed(Til.shape),
        ],
        out_specs=pl.BlockSpec((BN, E), lambda i: (i, 0)),
        out_shape=jax.ShapeDtypeStruct((N, E), jnp.float32),
    )(ur_sum, ea, base, Wbig, wT, M2, Sel, Sel2, Rep, Til)


def kernel(entity, edge_attr, entity_neighbours, entity_embeddings, u, W, w, M):
    N, K = entity_neighbours.shape
    V, R1, U = u.shape
    A = W.shape[2]
    E = entity_embeddings.shape[1]
    D = R1 * U

    HR = R1 // 2
    ub = lax.bitcast_convert_type(u, jnp.uint32)
    ub = (ub + 0x7FFF + ((ub >> 16) & 1)) >> 16
    u_pk = lax.bitcast_convert_type(
        ub[:, :HR, :] | (ub[:, HR:, :] << 16), jnp.int32).reshape(V, D // 2)
    nbr_flat = entity_neighbours.reshape(N * K)

    ur_sum, base = _sc_gather_sum(u_pk, nbr_flat, entity, entity_embeddings)

    Wbig = W.transpose(1, 0, 2).reshape(U, R1 * A) * (1.0 / K)
    M2 = M.reshape(R1 * U, E)
    eyeR = jnp.eye(R1, dtype=jnp.float32)
    eyeU = jnp.eye(U, dtype=jnp.float32)
    Sel = jnp.tile(eyeU, (R1, 1))
    Sel2 = jnp.repeat(eyeR, A, axis=0)
    Rep = Sel2.T
    Til = jnp.tile(eyeU, (1, R1))
    wT = w @ Til

    CH = 4
    NCk = N // CH
    outs = []
    for c in range(CH):
        sl = slice(c * NCk, (c + 1) * NCk)
        ur_c, base_c = _sc_gather_sum(
            u_pk, nbr_flat[c * NCk * K:(c + 1) * NCk * K],
            entity[sl], entity_embeddings)
        outs.append(_tc_dense(ur_c, edge_attr[sl], base_c, Wbig, wT, M2,
                              Sel, Sel2, Rep, Til, K, R1, U, A))
    return jnp.concatenate(outs, axis=0)


# speedup vs baseline: 2.3061x; 1.1082x over previous
import functools

import jax
import jax.numpy as jnp
from jax import lax
from jax.experimental import pallas as pl
from jax.experimental.pallas import tpu as pltpu
from jax.experimental.pallas import tpu_sc as plsc

NC = 2
NS = 16
NW = NC * NS


def _sc_gather_sum(u_pk, nbr_flat, entity, emb):
    V, DW = u_pk.shape
    N = entity.shape[0]
    K = nbr_flat.shape[0] // N
    E = emb.shape[1]
    TPW = N // NW
    BCH = 128
    NT = DW // 16

    mesh = plsc.VectorSubcoreMesh(
        core_axis_name="c", subcore_axis_name="s",
        num_cores=NC, num_subcores=NS)

    @functools.partial(
        pl.kernel,
        out_type=(jax.ShapeDtypeStruct((N, 2 * DW), jnp.float32),
                  jax.ShapeDtypeStruct((N, E), jnp.float32)),
        mesh=mesh,
        scratch_types=[
            pltpu.VMEM((TPW * K,), jnp.int32),
            pltpu.VMEM((TPW,), jnp.int32),
            pltpu.VMEM((4, K, DW), jnp.int32),
            pltpu.VMEM((2, 2, 2 * DW), jnp.float32),
            pltpu.VMEM((BCH, E), jnp.float32),
            pltpu.SemaphoreType.DMA,
            pltpu.SemaphoreType.DMA,
            pltpu.SemaphoreType.DMA,
            pltpu.SemaphoreType.DMA,
            pltpu.SemaphoreType.DMA,
            pltpu.SemaphoreType.DMA,
            pltpu.SemaphoreType.DMA,
        ],
    )
    def k(u_hbm, nbr_hbm, ent_hbm, emb_hbm, ur_hbm, base_hbm,
          idx_v, ent_v, rows_v, out_v, base_v, g0, g1, g2, g3,
          so0, so1, sb):
        wid = lax.axis_index("s") * NC + lax.axis_index("c")
        t0 = wid * TPW
        gsem = (g0, g1, g2, g3)
        osem = (so0, so1)

        pltpu.sync_copy(nbr_hbm.at[pl.ds(t0 * K, TPW * K)], idx_v)
        pltpu.sync_copy(ent_hbm.at[pl.ds(t0, TPW)], ent_v)

        def gather(lt, b):
            return pltpu.make_async_copy(
                u_hbm.at[idx_v.at[pl.ds(lt * K, K)]], rows_v.at[b], gsem[b])

        def out_dma(ob, row):
            return pltpu.make_async_copy(
                out_v.at[ob], ur_hbm.at[pl.ds(row, 2)], osem[ob])

        def load_pair(gb, kk, j):
            word = rows_v[gb, kk, pl.ds(j * 16, 16)]
            lo = lax.bitcast_convert_type(word << 16, jnp.float32)
            hi = lax.bitcast_convert_type(word, jnp.float32)
            return lo, hi

        def reduce_rows(gb, b, ob):
            def body(kk, acc):
                new = []
                for j in range(NT):
                    lo, hi = load_pair(gb, kk, j)
                    new.append(acc[2 * j] + lo)
                    new.append(acc[2 * j + 1] + hi)
                return tuple(new)
            acc = []
            for j in range(NT):
                lo, hi = load_pair(gb, 0, j)
                acc.extend((lo, hi))
            acc = lax.fori_loop(1, K, body, tuple(acc))
            for j in range(NT):
                out_v[ob, b, pl.ds(j * 16, 16)] = acc[2 * j]
                out_v[ob, b, pl.ds(DW + j * 16, 16)] = acc[2 * j + 1]

        gather(0, 0).start()
        gather(1, 1).start()
        gather(2, 2).start()

        def outer(cc, carry):
            for ob in range(2):
                @pl.when(cc > 0)
                def _wait_prev_out():
                    out_dma(ob, 0).wait()
                for b in range(2):
                    gb = ob * 2 + b
                    lt = cc * 4 + gb
                    nxt = jnp.minimum(lt + 3, TPW - 1)
                    gather(nxt, (gb + 3) % 4).start()
                    gather(lt, gb).wait()
                    reduce_rows(gb, b, ob)
                out_dma(ob, t0 + cc * 4 + ob * 2).start()
            return carry

        lax.fori_loop(0, TPW // 4, outer, 0)

        gather(0, 0).wait()
        gather(0, 1).wait()
        gather(0, 2).wait()
        out_dma(0, 0).wait()
        out_dma(1, 0).wait()

        for c in range(TPW // BCH):
            pltpu.async_copy(
                emb_hbm.at[ent_v.at[pl.ds(c * BCH, BCH)]], base_v, sb).wait()
            pltpu.sync_copy(base_v, base_hbm.at[pl.ds(t0 + c * BCH, BCH)])

    return k(u_pk, nbr_flat, entity, emb)


def _tc_dense(ur_sum, ea, base, Wbig, wT, M2, Sel, Sel2, Rep, Til,
              K, R1, U, A):
    N, D = ur_sum.shape
    E = base.shape[1]
    BN = 512
    inv_k = 1.0 / K

    def body(ur_ref, ea_ref, base_ref, Wbig_ref, wT_ref, M2_ref,
             Sel_ref, Sel2_ref, Rep_ref, Til_ref, out_ref):
        urb = ur_ref[...]
        eab = ea_ref[...]
        Wb = Wbig_ref[...]
        Selm = Sel_ref[...]
        earep = jnp.dot(eab, Rep_ref[...],
                        preferred_element_type=jnp.float32)
        wrrep = jnp.dot(eab, wT_ref[...],
                        preferred_element_type=jnp.float32)

        qs = []
        for r in range(R1):
            ur_r = urb[:, r * U:(r + 1) * U]
            Pr = jnp.dot(ur_r, Wb,
                         preferred_element_type=jnp.float32)
            Qr = jnp.dot(Pr * earep, Selm,
                         preferred_element_type=jnp.float32)
            qs.append(Qr)
        tq = jnp.tanh(jnp.concatenate(qs, axis=1))

        scores = jnp.dot(tq * wrrep, Sel2_ref[...],
                         preferred_element_type=jnp.float32)
        m = jnp.max(scores, axis=1, keepdims=True)
        ex = jnp.exp(scores - m)
        att = ex / jnp.sum(ex, axis=1, keepdims=True)
        attk = att * inv_k

        attrep = jnp.dot(attk, Rep_ref[...],
                         preferred_element_type=jnp.float32)
        vv = jnp.dot(attrep * urb, Selm,
                     preferred_element_type=jnp.float32)
        T = jnp.dot(vv, Til_ref[...],
                    preferred_element_type=jnp.float32) * earep
        agg = jnp.dot(T, M2_ref[...],
                      preferred_element_type=jnp.float32)
        out = base_ref[...] + agg
        nrm = jnp.sqrt(jnp.sum(out * out, axis=1, keepdims=True))
        out_ref[...] = out / jnp.maximum(nrm, 1e-12)

    grid = (N // BN,)
    fixed = lambda shape: pl.BlockSpec(shape, lambda i: (0, 0))
    return pl.pallas_call(
        body,
        grid=grid,
        in_specs=[
            pl.BlockSpec((BN, D), lambda i: (i, 0)),
            pl.BlockSpec((BN, R1), lambda i: (i, 0)),
            pl.BlockSpec((BN, E), lambda i: (i, 0)),
            fixed(Wbig.shape),
            fixed(wT.shape),
            fixed(M2.shape),
            fixed(Sel.shape),
            fixed(Sel2.shape),
            fixed(Rep.shape),
            fixed(Til.shape),
        ],
        out_specs=pl.BlockSpec((BN, E), lambda i: (i, 0)),
        out_shape=jax.ShapeDtypeStruct((N, E), jnp.float32),
    )(ur_sum, ea, base, Wbig, wT, M2, Sel, Sel2, Rep, Til)


def kernel(entity, edge_attr, entity_neighbours, entity_embeddings, u, W, w, M):
    N, K = entity_neighbours.shape
    V, R1, U = u.shape
    A = W.shape[2]
    E = entity_embeddings.shape[1]
    D = R1 * U

    HR = R1 // 2
    lo = lax.bitcast_convert_type(
        u[:, :HR, :].astype(jnp.bfloat16), jnp.uint16).astype(jnp.uint32)
    hi = lax.bitcast_convert_type(
        u[:, HR:, :].astype(jnp.bfloat16), jnp.uint16).astype(jnp.uint32)
    u_pk = lax.bitcast_convert_type(lo | (hi << 16),
                                    jnp.int32).reshape(V, D // 2)
    nbr_flat = entity_neighbours.reshape(N * K)

    ur_sum, base = _sc_gather_sum(u_pk, nbr_flat, entity, entity_embeddings)

    Wbig = W.transpose(1, 0, 2).reshape(U, R1 * A) * (1.0 / K)
    M2 = M.reshape(R1 * U, E)
    eyeR = jnp.eye(R1, dtype=jnp.float32)
    eyeU = jnp.eye(U, dtype=jnp.float32)
    Sel = jnp.tile(eyeU, (R1, 1))
    Sel2 = jnp.repeat(eyeR, A, axis=0)
    Rep = Sel2.T
    Til = jnp.tile(eyeU, (1, R1))
    wT = w @ Til

    CH = 4
    NCk = N // CH
    outs = []
    for c in range(CH):
        sl = slice(c * NCk, (c + 1) * NCk)
        ur_c, base_c = _sc_gather_sum(
            u_pk, nbr_flat[c * NCk * K:(c + 1) * NCk * K],
            entity[sl], entity_embeddings)
        outs.append(_tc_dense(ur_c, edge_attr[sl], base_c, Wbig, wT, M2,
                              Sel, Sel2, Rep, Til, K, R1, U, A))
    return jnp.concatenate(outs, axis=0)
